# Initial kernel scaffold; baseline (speedup 1.0000x reference)
import jax
import jax.numpy as jnp
from jax.experimental import pallas as pl


def kernel(x, t, tables):
    raise NotImplementedError("write your pallas kernel here")


# speedup vs baseline: 186.3487x; 186.3487x over previous
import functools

import jax
import jax.numpy as jnp
import numpy as np
from jax import lax
from jax.experimental import pallas as pl
from jax.experimental.pallas import tpu as pltpu
from jax.experimental.pallas import tpu_sc as plsc

TIME_RES = 8
NL = 8
F = 4
H = 1 << 14
NB = 4
N_PTS = 1048576
_PLS = float(np.exp2(np.log2(32768 / 512) / (NL - 1)))
SCALES = np.array(
    [np.exp2(l * np.log2(_PLS)) * 512 - 1.0 for l in range(NL)], dtype=np.float32
)
PRIME1 = np.uint32(2654435761)
HMASK = np.uint32(H - 1)

NC = 2
NS = 16
LANES = 16
NW = NC * NS

NCHUNK = NW // NL
CHUNK_PTS = N_PTS // NCHUNK
PB = 8192
NKB = CHUNK_PTS // PB
CH = 2048
CROW = 64


def _sc_body(s1_hbm, s2_hbm, const_hbm, x_hbm, out_hbm,
             comb_v, s1buf, s2buf, pbuf, cbuf, xsbuf, ysbuf, obuf):
    cid = lax.axis_index("c")
    sid = lax.axis_index("s")
    wid = sid * NC + cid
    level = wid % NL
    chunk = wid // NL

    pltpu.sync_copy(const_hbm.at[pl.ds(wid * CROW, CROW)], cbuf)
    pat1 = cbuf[pl.ds(0, LANES)]
    pat2 = cbuf[pl.ds(LANES, LANES)]
    scale = cbuf[pl.ds(2 * LANES, LANES)]
    iota = lax.iota(jnp.int32, LANES)

    lhf = level * (H * F)

    def stage_a(ci, carry):
        off = lhf + ci * (CH * F)
        pltpu.sync_copy(s1_hbm.at[pl.ds(off, CH * F)], s1buf)
        pltpu.sync_copy(s2_hbm.at[pl.ds(off, CH * F)], s2buf)

        def premul(g, c_):
            s = pl.ds(g * LANES, LANES)
            pbuf[s] = s1buf[s] * pat1 + s2buf[s] * pat2
            return c_

        lax.fori_loop(0, CH * F // LANES, premul, carry)

        def reduce4(g, c_):
            idx = g * (LANES * F) + iota * F
            acc = plsc.load_gather(pbuf, [idx])
            acc = acc + plsc.load_gather(pbuf, [idx + 1])
            acc = acc + plsc.load_gather(pbuf, [idx + 2])
            acc = acc + plsc.load_gather(pbuf, [idx + 3])
            comb_v[pl.ds(ci * CH + g * LANES, LANES)] = acc
            return c_

        return lax.fori_loop(0, CH // LANES, reduce4, carry)

    lax.fori_loop(0, H // CH, stage_a, 0)

    pbase = chunk * CHUNK_PTS

    def stage_b(k, carry):
        xoff = pbase + k * PB
        pltpu.sync_copy(x_hbm.at[pl.ds(xoff, PB)], xsbuf)
        pltpu.sync_copy(x_hbm.at[pl.ds(N_PTS + xoff, PB)], ysbuf)

        def inner(g, c_):
            xs = xsbuf[pl.ds(g * LANES, LANES)]
            ys = ysbuf[pl.ds(g * LANES, LANES)]
            px = xs * scale + 0.5
            py = ys * scale + 0.5
            ix = px.astype(jnp.int32)
            iy = py.astype(jnp.int32)
            wx = px - ix.astype(jnp.float32)
            wy = py - iy.astype(jnp.float32)
            ux = ix.astype(jnp.uint32)
            uy = iy.astype(jnp.uint32)
            hy0 = uy * PRIME1
            hy1 = hy0 + PRIME1
            ux1 = ux + np.uint32(1)
            h00 = ((ux ^ hy0) & HMASK).astype(jnp.int32)
            h10 = ((ux1 ^ hy0) & HMASK).astype(jnp.int32)
            h01 = ((ux ^ hy1) & HMASK).astype(jnp.int32)
            h11 = ((ux1 ^ hy1) & HMASK).astype(jnp.int32)
            g00 = plsc.load_gather(comb_v, [h00])
            g10 = plsc.load_gather(comb_v, [h10])
            g01 = plsc.load_gather(comb_v, [h01])
            g11 = plsc.load_gather(comb_v, [h11])
            gx0 = g00 + (g10 - g00) * wx
            gx1 = g01 + (g11 - g01) * wx
            res = gx0 + (gx1 - gx0) * wy
            obuf[pl.ds(g * LANES, LANES)] = res
            return c_

        lax.fori_loop(0, PB // LANES, inner, carry)
        pltpu.sync_copy(obuf, out_hbm.at[pl.ds(level * N_PTS + xoff, PB)])
        return carry

    lax.fori_loop(0, NKB, stage_b, 0)


_sc_call = functools.partial(
    pl.kernel,
    out_type=jax.ShapeDtypeStruct((NL * N_PTS,), jnp.float32),
    mesh=plsc.VectorSubcoreMesh(
        core_axis_name="c", subcore_axis_name="s", num_cores=NC, num_subcores=NS
    ),
    compiler_params=pltpu.CompilerParams(needs_layout_passes=False),
    scratch_types=[
        pltpu.VMEM((H,), jnp.float32),
        pltpu.VMEM((CH * F,), jnp.float32),
        pltpu.VMEM((CH * F,), jnp.float32),
        pltpu.VMEM((CH * F,), jnp.float32),
        pltpu.VMEM((CROW,), jnp.float32),
        pltpu.VMEM((PB,), jnp.float32),
        pltpu.VMEM((PB,), jnp.float32),
        pltpu.VMEM((PB,), jnp.float32),
    ],
)(_sc_body)


def kernel(x, t, tables):
    idx = t * (TIME_RES - 1)
    i1 = jnp.floor(idx).astype(jnp.int32)
    i2 = jnp.ceil(idx).astype(jnp.int32)
    same = i1 == i2
    w1 = jnp.where(same, jnp.float32(1.0), i2.astype(jnp.float32) - idx)
    w2 = jnp.where(same, jnp.float32(0.0), idx - i1.astype(jnp.float32))
    Tm = [i / (NB - 1) for i in range(NB)]
    bs = []
    for j in range(NB):
        b = jnp.float32(1.0)
        for m in range(NB):
            if m != j:
                b = b * (t - Tm[m]) / (Tm[j] - Tm[m])
        bs.append(b)
    b = jnp.stack(bs)

    pat1 = jnp.tile(w1 * b, F)
    pat2 = jnp.tile(w2 * b, F)
    lvl = jnp.arange(NW, dtype=jnp.int32) % NL
    scal = jnp.asarray(SCALES)[lvl]
    const_rows = jnp.concatenate(
        [
            jnp.broadcast_to(pat1, (NW, LANES)),
            jnp.broadcast_to(pat2, (NW, LANES)),
            jnp.broadcast_to(scal[:, None], (NW, LANES)),
            jnp.zeros((NW, CROW - 3 * LANES), jnp.float32),
        ],
        axis=1,
    ).reshape(-1)

    slab1 = jnp.take(tables, i1, axis=0).reshape(-1)
    slab2 = jnp.take(tables, i2, axis=0).reshape(-1)
    xflat = x.T.reshape(-1)

    out_flat = _sc_call(slab1, slab2, const_rows, xflat)
    return out_flat.reshape(NL, N_PTS).T
